# baseline (speedup 1.0000x reference)
import functools
import math

import jax
import jax.numpy as jnp
from jax import lax
from jax.experimental import pallas as pl
from jax.experimental.pallas import tpu as pltpu
from jax.experimental.pallas import tpu_sc as plsc

_B, _L, _D, _V = 4096, 200, 64, 1000000
_DP = 128
_NC, _NS = 2, 16
_NW = _NC * _NS
_BB = _B // _NW


def _pe_body(out_ref):
    row = lax.broadcasted_iota(jnp.int32, (_L, _DP), 0).astype(jnp.float32)
    col = lax.broadcasted_iota(jnp.int32, (_L, _DP), 1)
    expo = (col // 2).astype(jnp.float32) * (2.0 / _D)
    denom = jnp.exp(expo * math.log(10000.0))
    angle = row / denom
    out_ref[...] = jnp.where(col % 2 == 0, jnp.sin(angle), jnp.cos(angle))


def _make_pe():
    return pl.pallas_call(
        _pe_body,
        out_shape=jax.ShapeDtypeStruct((_L, _DP), jnp.float32),
    )()


_sc_mesh = plsc.VectorSubcoreMesh(core_axis_name="c", subcore_axis_name="s")


@functools.partial(
    pl.kernel,
    out_type=jax.ShapeDtypeStruct((_L, _D, _B), jnp.float32),
    mesh=_sc_mesh,
    scratch_types=[
        pltpu.VMEM((_L, _BB), jnp.int32),
        pltpu.VMEM((2, _BB), jnp.int32),
        pltpu.VMEM((2, _BB, _DP), jnp.float32),
        pltpu.VMEM((2, _D, _BB), jnp.float32),
        pltpu.VMEM((_L, _DP), jnp.float32),
        pltpu.SemaphoreType.DMA((2,)),
        pltpu.SemaphoreType.DMA((2,)),
    ],
    compiler_params=pltpu.CompilerParams(
        use_tc_tiling_on_sc=True, needs_layout_passes=False),
)
def _sc_embed(tokt_hbm, pe_hbm, tab_hbm, out_hbm, tokbuf, idxstg, gbuf,
              tbuf, pe_v, gsem, osem):
    wid = lax.axis_index("s") * _NC + lax.axis_index("c")
    b0 = wid * _BB

    pltpu.sync_copy(tokt_hbm.at[:, pl.ds(b0, _BB)], tokbuf)
    pltpu.sync_copy(pe_hbm, pe_v)

    iotas = [lax.iota(jnp.int32, 16) + jj * 16 for jj in range(_BB // 16)]

    def prep_idx(l, b):
        for jj in range(_BB // 16):
            t = tokbuf[l, pl.ds(jj * 16, 16)]
            idxstg[b, pl.ds(jj * 16, 16)] = lax.shift_right_logical(t, 1)

    def start_gather(b):
        pltpu.async_copy(tab_hbm.at[idxstg.at[b]], gbuf.at[b], gsem.at[b])

    def wait_gather(b):
        pltpu.make_async_copy(
            tab_hbm.at[pl.ds(0, _BB)], gbuf.at[b], gsem.at[b]
        ).wait()

    def start_out(l, b):
        pltpu.async_copy(
            tbuf.at[b], out_hbm.at[l, :, pl.ds(b0, _BB)], osem.at[b]
        )

    def wait_out(b):
        pltpu.make_async_copy(
            tbuf.at[b], out_hbm.at[0, :, pl.ds(0, _BB)], osem.at[b]
        ).wait()

    prep_idx(0, 0)
    start_gather(0)

    def body(l, carry):
        b = l % 2
        nb = 1 - b

        @pl.when(l + 1 < _L)
        def _():
            prep_idx(l + 1, nb)

            @pl.when(l >= 1)
            def _():
                wait_out(nb)
            start_gather(nb)

        wait_gather(b)

        bsplat = b + jnp.zeros((16,), jnp.int32)
        lsplat = l + jnp.zeros((16,), jnp.int32)
        pars = []
        for jj in range(_BB // 16):
            t = tokbuf[l, pl.ds(jj * 16, 16)]
            pars.append(lax.shift_left(t & 1, 6))

        def dloop(d, c2):
            dsplat = d + jnp.zeros((16,), jnp.int32)
            pev = plsc.load_gather(pe_v, [lsplat, dsplat])
            for jj in range(_BB // 16):
                colidx = pars[jj] + d
                v = plsc.load_gather(gbuf, [bsplat, iotas[jj], colidx])
                tbuf[b, d, pl.ds(jj * 16, 16)] = v + pev
            return c2

        lax.fori_loop(0, _D, dloop, 0)
        start_out(l, b)
        return carry

    lax.fori_loop(0, _L, body, 0)
    wait_out(0)
    wait_out(1)


def kernel(tokens, table):
    tokt = jnp.transpose(tokens.astype(jnp.int32))
    tabv = table.reshape(_V // 2, _DP)
    pe = _make_pe()
    out_t = _sc_embed(tokt, pe, tabv)
    return jnp.transpose(out_t, (2, 0, 1))


# speedup vs baseline: 1.1743x; 1.1743x over previous
import functools
import math

import jax
import jax.numpy as jnp
from jax import lax
from jax.experimental import pallas as pl
from jax.experimental.pallas import tpu as pltpu
from jax.experimental.pallas import tpu_sc as plsc

_B, _L, _D, _V = 4096, 200, 64, 1000000
_DP = 128
_TP = 132
_NC, _NS = 2, 16
_NW = _NC * _NS
_BB = _B // _NW
_JU = 8


def _pe_body(out_ref):
    row = lax.broadcasted_iota(jnp.int32, (_L, _DP), 0).astype(jnp.float32)
    col = lax.broadcasted_iota(jnp.int32, (_L, _DP), 1)
    expo = (col // 2).astype(jnp.float32) * (2.0 / _D)
    denom = jnp.exp(expo * math.log(10000.0))
    angle = row / denom
    out_ref[...] = jnp.where(col % 2 == 0, jnp.sin(angle), jnp.cos(angle))


def _make_pe():
    return pl.pallas_call(
        _pe_body,
        out_shape=jax.ShapeDtypeStruct((_L, _DP), jnp.float32),
    )()


_sc_mesh = plsc.VectorSubcoreMesh(core_axis_name="c", subcore_axis_name="s")


@functools.partial(
    pl.kernel,
    out_type=jax.ShapeDtypeStruct((_L, _D, _B), jnp.float32),
    mesh=_sc_mesh,
    scratch_types=[
        pltpu.VMEM((_L, _BB), jnp.int32),
        pltpu.VMEM((2, _BB, _DP), jnp.float32),
        pltpu.VMEM((2, _D, _TP), jnp.float32),
        pltpu.VMEM((_L, _DP), jnp.float32),
        pltpu.SemaphoreType.DMA((2,)),
        pltpu.SemaphoreType.DMA((2,)),
    ],
    compiler_params=pltpu.CompilerParams(
        use_tc_tiling_on_sc=True, needs_layout_passes=False),
)
def _sc_embed(tokt_hbm, pe_hbm, tab_hbm, out_hbm, tokbuf, gbuf, tbuf, pe_v,
              gsem, osem):
    wid = lax.axis_index("s") * _NC + lax.axis_index("c")
    b0 = wid * _BB

    pltpu.sync_copy(tokt_hbm.at[:, pl.ds(b0, _BB)], tokbuf)
    pltpu.sync_copy(pe_hbm, pe_v)

    def start_gather(l, b):
        pltpu.async_copy(tab_hbm.at[tokbuf.at[l]], gbuf.at[b], gsem.at[b])

    def wait_gather(b):
        pltpu.make_async_copy(
            tab_hbm.at[pl.ds(0, _BB)], gbuf.at[b], gsem.at[b]
        ).wait()

    def start_out(l, b):
        pltpu.async_copy(
            tbuf.at[b].at[:, pl.ds(0, _BB)],
            out_hbm.at[l, :, pl.ds(b0, _BB)],
            osem.at[b],
        )

    def wait_out(b):
        pltpu.make_async_copy(
            tbuf.at[b].at[:, pl.ds(0, _BB)],
            out_hbm.at[0, :, pl.ds(0, _BB)],
            osem.at[b],
        ).wait()

    iota = lax.iota(jnp.int32, 16)
    rowidx = [iota + c * 16 for c in range(_D // 16)]

    start_gather(0, 0)

    def body(l, carry):
        b = l % 2
        nb = 1 - b

        @pl.when(l + 1 < _L)
        def _():
            @pl.when(l >= 1)
            def _():
                wait_out(nb)
            start_gather(l + 1, nb)

        wait_gather(b)

        pev = [pe_v[l, pl.ds(c * 16, 16)] for c in range(_D // 16)]
        tb = tbuf.at[b]
        gb = gbuf.at[b]

        def jloop(jb, c2):
            j0 = jb * _JU
            for jj in range(_JU):
                j = j0 + jj
                jsplat = j + jnp.zeros((16,), jnp.int32)
                for c in range(_D // 16):
                    v = gb[j, pl.ds(c * 16, 16)] + pev[c]
                    plsc.store_scatter(tb, [rowidx[c], jsplat], v)
            return c2

        lax.fori_loop(0, _BB // _JU, jloop, 0)
        start_out(l, b)
        return carry

    lax.fori_loop(0, _L, body, 0)
    wait_out(0)
    wait_out(1)


def kernel(tokens, table):
    tokt = jnp.transpose(tokens.astype(jnp.int32))
    tab128 = jnp.pad(table, ((0, 0), (0, _DP - _D)))
    pe = _make_pe()
    out_t = _sc_embed(tokt, pe, tab128)
    return jnp.transpose(out_t, (2, 0, 1))


# speedup vs baseline: 1.6163x; 1.3764x over previous
import functools
import math

import jax
import jax.numpy as jnp
from jax import lax
from jax.experimental import pallas as pl
from jax.experimental.pallas import tpu as pltpu
from jax.experimental.pallas import tpu_sc as plsc

_B, _L, _D, _V = 4096, 200, 64, 1000000
_NC, _NS = 2, 16
_NW = _NC * _NS
_SEQ_W = _B // _NW
_ROWS_W = _SEQ_W * _L
_C = 2 * _L
_NG = _ROWS_W // _C
_IPG = 100
_PPC = _C // _IPG
_IDX_ROWS = _ROWS_W // _IPG


def _pe_body(out_ref):
    row = lax.broadcasted_iota(jnp.int32, (_L, _D), 0).astype(jnp.float32)
    col = lax.broadcasted_iota(jnp.int32, (_L, _D), 1)
    expo = (col // 2).astype(jnp.float32) * (2.0 / _D)
    denom = jnp.exp(expo * math.log(10000.0))
    angle = row / denom
    out_ref[...] = jnp.where(col % 2 == 0, jnp.sin(angle), jnp.cos(angle))


def _make_pe():
    return pl.pallas_call(
        _pe_body,
        out_shape=jax.ShapeDtypeStruct((_L, _D), jnp.float32),
    )()


_sc_mesh = plsc.VectorSubcoreMesh(core_axis_name="c", subcore_axis_name="s")


@functools.partial(
    pl.kernel,
    out_type=jax.ShapeDtypeStruct((_B * _L, _D), jnp.float32),
    mesh=_sc_mesh,
    scratch_types=[
        pltpu.VMEM((_IDX_ROWS, _IPG), jnp.int32),
        pltpu.VMEM((2, _C, _D), jnp.float32),
        pltpu.VMEM_SHARED((_C, _D), jnp.float32),
        pltpu.SemaphoreType.DMA((2,)),
        pltpu.SemaphoreType.DMA((2,)),
    ],
    compiler_params=pltpu.CompilerParams(use_tc_tiling_on_sc=False),
)
def _sc_embed(tok_hbm, pe_hbm, table_hbm, out_hbm, idx_v, rows_v, pe2_sh,
              gsem, osem):
    sid = lax.axis_index("s")
    wid = sid * _NC + lax.axis_index("c")
    row0 = wid * _ROWS_W
    irow0 = wid * _IDX_ROWS

    pltpu.sync_copy(tok_hbm.at[pl.ds(irow0, _IDX_ROWS)], idx_v)

    @pl.when(sid == 0)
    def _():
        pltpu.sync_copy(pe_hbm, pe2_sh.at[pl.ds(0, _L)])
        pltpu.sync_copy(pe_hbm, pe2_sh.at[pl.ds(_L, _L)])
    plsc.subcore_barrier()

    def prefill(b):
        pltpu.sync_copy(pe2_sh, rows_v.at[b])

    def start_gather(g, b):
        for p in range(_PPC):
            pltpu.async_copy(
                table_hbm.at[idx_v.at[g * _PPC + p]],
                rows_v.at[b].at[pl.ds(p * _IPG, _IPG)],
                gsem.at[b],
                add=True,
            )

    def wait_gather(b):
        pltpu.make_async_copy(
            out_hbm.at[pl.ds(0, _C)], rows_v.at[b], gsem.at[b]
        ).wait()

    def start_out(g, b):
        pltpu.async_copy(
            rows_v.at[b], out_hbm.at[pl.ds(row0 + g * _C, _C)], osem.at[b]
        )

    def wait_out(b):
        pltpu.make_async_copy(
            rows_v.at[b], out_hbm.at[pl.ds(0, _C)], osem.at[b]
        ).wait()

    prefill(0)
    start_gather(0, 0)

    def body(g, carry):
        b = g % 2
        nb = 1 - b

        @pl.when(g + 1 < _NG)
        def _():
            @pl.when(g >= 1)
            def _():
                wait_out(nb)
            prefill(nb)
            start_gather(g + 1, nb)

        wait_gather(b)
        start_out(g, b)
        return carry

    lax.fori_loop(0, _NG, body, 0)
    wait_out(0)
    wait_out(1)


def kernel(tokens, table):
    tok = tokens.reshape(-1).astype(jnp.int32).reshape(_B * _L // _IPG, _IPG)
    pe = _make_pe()
    out = _sc_embed(tok, pe, table)
    return out.reshape(_B, _L, _D)
